# Initial kernel scaffold; baseline (speedup 1.0000x reference)
import jax
import jax.numpy as jnp
from jax.experimental import pallas as pl


def kernel(weight):
    raise NotImplementedError("write your pallas kernel here")


# speedup vs baseline: 2.5078x; 2.5078x over previous
import jax
import jax.numpy as jnp
from jax.experimental import pallas as pl
from jax.experimental.pallas import tpu as pltpu

N = 8192
EPS = 1e-8
BR_IO = 256
BR = 512
VMEM_LIMIT = 56 * 1024 * 1024


def _first_kernel(w_ref, k_ref, un_ref, vn_ref, t_ref):
    i = pl.program_id(0)

    @pl.when(i == 0)
    def _():
        t_ref[...] = jnp.zeros_like(t_ref)

    kb = jnp.exp(w_ref[...])
    k_ref[...] = kb
    s = jnp.sum(kb, axis=1, keepdims=True)
    un = 1.0 / (s + EPS)
    un_ref[...] = jnp.broadcast_to(un, (BR_IO, 128))
    t_ref[...] += jnp.sum(kb * un, axis=0, keepdims=True)

    @pl.when(i == N // BR_IO - 1)
    def _():
        vn_ref[...] = 1.0 / (t_ref[...] + EPS)


def _iter_kernel(k_ref, u_ref, v_ref, un_ref, vn_ref, t_ref):
    i = pl.program_id(0)

    @pl.when(i == 0)
    def _():
        t_ref[...] = jnp.zeros_like(t_ref)

    kb = k_ref[...]
    v = v_ref[...]
    s = jnp.sum(kb * v, axis=1, keepdims=True)
    u = u_ref[:, 0:1]
    un = u / (u * s + EPS)
    un_ref[...] = jnp.broadcast_to(un, (BR, 128))
    t_ref[...] += jnp.sum(kb * un, axis=0, keepdims=True)

    @pl.when(i == N // BR - 1)
    def _():
        vv = v_ref[...]
        vn_ref[...] = vv / (vv * t_ref[...] + EPS)


def _final_kernel(k_ref, u_ref, v_ref, p_ref):
    p_ref[...] = k_ref[...] * u_ref[:, 0:1] * v_ref[...]


def kernel(weight):
    f32 = jnp.float32
    nb_io = N // BR_IO
    nb = N // BR

    k_mat, u, v = pl.pallas_call(
        _first_kernel,
        grid=(nb_io,),
        in_specs=[pl.BlockSpec((BR_IO, N), lambda i: (i, 0))],
        out_specs=[
            pl.BlockSpec((BR_IO, N), lambda i: (i, 0)),
            pl.BlockSpec((BR_IO, 128), lambda i: (i, 0)),
            pl.BlockSpec((1, N), lambda i: (0, 0)),
        ],
        out_shape=[
            jax.ShapeDtypeStruct((N, N), f32),
            jax.ShapeDtypeStruct((N, 128), f32),
            jax.ShapeDtypeStruct((1, N), f32),
        ],
        scratch_shapes=[pltpu.VMEM((1, N), f32)],
        compiler_params=pltpu.CompilerParams(
            dimension_semantics=("arbitrary",),
            vmem_limit_bytes=VMEM_LIMIT,
        ),
        name="sinkhorn_first",
    )(weight)

    iter_call = pl.pallas_call(
        _iter_kernel,
        grid=(nb,),
        in_specs=[
            pl.BlockSpec((BR, N), lambda i: (i, 0)),
            pl.BlockSpec((BR, 128), lambda i: (i, 0)),
            pl.BlockSpec((1, N), lambda i: (0, 0)),
        ],
        out_specs=[
            pl.BlockSpec((BR, 128), lambda i: (i, 0)),
            pl.BlockSpec((1, N), lambda i: (0, 0)),
        ],
        out_shape=[
            jax.ShapeDtypeStruct((N, 128), f32),
            jax.ShapeDtypeStruct((1, N), f32),
        ],
        scratch_shapes=[pltpu.VMEM((1, N), f32)],
        compiler_params=pltpu.CompilerParams(
            dimension_semantics=("arbitrary",),
            vmem_limit_bytes=VMEM_LIMIT,
        ),
        name="sinkhorn_iter",
    )

    for _ in range(9):
        u, v = iter_call(k_mat, u, v)

    return pl.pallas_call(
        _final_kernel,
        grid=(nb_io,),
        in_specs=[
            pl.BlockSpec((BR_IO, N), lambda i: (i, 0)),
            pl.BlockSpec((BR_IO, 128), lambda i: (i, 0)),
            pl.BlockSpec((1, N), lambda i: (0, 0)),
        ],
        out_specs=pl.BlockSpec((BR_IO, N), lambda i: (i, 0)),
        out_shape=jax.ShapeDtypeStruct((N, N), f32),
        compiler_params=pltpu.CompilerParams(
            dimension_semantics=("arbitrary",),
            vmem_limit_bytes=VMEM_LIMIT,
        ),
        name="sinkhorn_final",
    )(k_mat, u, v)


# speedup vs baseline: 2.9997x; 1.1962x over previous
import jax
import jax.numpy as jnp
from jax.experimental import pallas as pl
from jax.experimental.pallas import tpu as pltpu

N = 8192
EPS = 1e-8
BR_IO = 256
BR = 512
VMEM_LIMIT = 56 * 1024 * 1024


def _first_kernel(w_ref, k_ref, un_ref, vn_ref, t_ref):
    i = pl.program_id(0)

    @pl.when(i == 0)
    def _():
        t_ref[...] = jnp.zeros_like(t_ref)

    kb = jnp.exp(w_ref[...])
    k_ref[...] = kb.astype(jnp.bfloat16)
    s = jnp.sum(kb, axis=1, keepdims=True)
    un = 1.0 / (s + EPS)
    un_ref[...] = jnp.broadcast_to(un, (BR_IO, 128))
    t_ref[...] += jnp.sum(kb * un, axis=0, keepdims=True)

    @pl.when(i == N // BR_IO - 1)
    def _():
        vn_ref[...] = 1.0 / (t_ref[...] + EPS)


def _iter_kernel(k_ref, u_ref, v_ref, un_ref, vn_ref, t_ref):
    i = pl.program_id(0)

    @pl.when(i == 0)
    def _():
        t_ref[...] = jnp.zeros_like(t_ref)

    kb = k_ref[...].astype(jnp.float32)
    v = v_ref[...]
    s = jnp.sum(kb * v, axis=1, keepdims=True)
    u = u_ref[:, 0:1]
    un = u / (u * s + EPS)
    un_ref[...] = jnp.broadcast_to(un, (BR, 128))
    t_ref[...] += jnp.sum(kb * un, axis=0, keepdims=True)

    @pl.when(i == N // BR - 1)
    def _():
        vv = v_ref[...]
        vn_ref[...] = vv / (vv * t_ref[...] + EPS)


def _final_kernel(k_ref, u_ref, v_ref, p_ref):
    p_ref[...] = (
        k_ref[...].astype(jnp.float32) * u_ref[:, 0:1] * v_ref[...]
    )


def kernel(weight):
    f32 = jnp.float32
    nb_io = N // BR_IO
    nb = N // BR

    k_mat, u, v = pl.pallas_call(
        _first_kernel,
        grid=(nb_io,),
        in_specs=[pl.BlockSpec((BR_IO, N), lambda i: (i, 0))],
        out_specs=[
            pl.BlockSpec((BR_IO, N), lambda i: (i, 0)),
            pl.BlockSpec((BR_IO, 128), lambda i: (i, 0)),
            pl.BlockSpec((1, N), lambda i: (0, 0)),
        ],
        out_shape=[
            jax.ShapeDtypeStruct((N, N), jnp.bfloat16),
            jax.ShapeDtypeStruct((N, 128), f32),
            jax.ShapeDtypeStruct((1, N), f32),
        ],
        scratch_shapes=[pltpu.VMEM((1, N), f32)],
        compiler_params=pltpu.CompilerParams(
            dimension_semantics=("arbitrary",),
            vmem_limit_bytes=VMEM_LIMIT,
        ),
        name="sinkhorn_first",
    )(weight)

    iter_call = pl.pallas_call(
        _iter_kernel,
        grid=(nb,),
        in_specs=[
            pl.BlockSpec((BR, N), lambda i: (i, 0)),
            pl.BlockSpec((BR, 128), lambda i: (i, 0)),
            pl.BlockSpec((1, N), lambda i: (0, 0)),
        ],
        out_specs=[
            pl.BlockSpec((BR, 128), lambda i: (i, 0)),
            pl.BlockSpec((1, N), lambda i: (0, 0)),
        ],
        out_shape=[
            jax.ShapeDtypeStruct((N, 128), f32),
            jax.ShapeDtypeStruct((1, N), f32),
        ],
        scratch_shapes=[pltpu.VMEM((1, N), f32)],
        compiler_params=pltpu.CompilerParams(
            dimension_semantics=("arbitrary",),
            vmem_limit_bytes=VMEM_LIMIT,
        ),
        name="sinkhorn_iter",
    )

    for _ in range(9):
        u, v = iter_call(k_mat, u, v)

    return pl.pallas_call(
        _final_kernel,
        grid=(nb_io,),
        in_specs=[
            pl.BlockSpec((BR_IO, N), lambda i: (i, 0)),
            pl.BlockSpec((BR_IO, 128), lambda i: (i, 0)),
            pl.BlockSpec((1, N), lambda i: (0, 0)),
        ],
        out_specs=pl.BlockSpec((BR_IO, N), lambda i: (i, 0)),
        out_shape=jax.ShapeDtypeStruct((N, N), f32),
        scratch_shapes=[],
        compiler_params=pltpu.CompilerParams(
            dimension_semantics=("arbitrary",),
            vmem_limit_bytes=VMEM_LIMIT,
        ),
        name="sinkhorn_final",
    )(k_mat, u, v)


# speedup vs baseline: 3.9536x; 1.3180x over previous
import jax
import jax.numpy as jnp
from jax.experimental import pallas as pl
from jax.experimental.pallas import tpu as pltpu

N = 8192
EPS = 1e-8
BR_IO = 256
BR = 512
VMEM_LIMIT = 56 * 1024 * 1024


def _first_kernel(w_ref, k_ref, un_ref, vn_ref, t_ref):
    i = pl.program_id(0)

    @pl.when(i == 0)
    def _():
        t_ref[...] = jnp.zeros_like(t_ref)

    kb = jnp.exp(w_ref[...])
    k_ref[...] = kb.astype(jnp.bfloat16)
    s = jnp.sum(kb, axis=1, keepdims=True)
    un = 1.0 / (s + EPS)
    un_ref[...] = jnp.broadcast_to(un, (BR_IO, 128))
    t_ref[...] += jnp.sum(kb * un, axis=0, keepdims=True)

    @pl.when(i == N // BR_IO - 1)
    def _():
        vn_ref[...] = 1.0 / (t_ref[...] + EPS)


def _iter_kernel(k_ref, u_ref, v_ref, un_ref, vn_ref, t_ref):
    i = pl.program_id(0)

    @pl.when(i == 0)
    def _():
        t_ref[...] = jnp.zeros_like(t_ref)

    kb16 = k_ref[...]
    v16 = v_ref[...].astype(jnp.bfloat16)
    r = kb16 * v16
    r = r[:, : N // 2] + r[:, N // 2 :]
    r = r[:, : N // 4] + r[:, N // 4 :]
    r = r[:, : N // 8] + r[:, N // 8 :]
    s = jnp.sum(r.astype(jnp.float32), axis=1, keepdims=True)
    u = u_ref[:, 0:1]
    un = u / (u * s + EPS)
    un_ref[...] = jnp.broadcast_to(un, (BR, 128))
    x = kb16 * un.astype(jnp.bfloat16)
    x = x[: BR // 2] + x[BR // 2 :]
    x = x[: BR // 4] + x[BR // 4 :]
    x = x[: BR // 8] + x[BR // 8 :]
    t_ref[...] += jnp.sum(
        x.astype(jnp.float32), axis=0, keepdims=True
    )

    @pl.when(i == N // BR - 1)
    def _():
        vv = v_ref[...]
        vn_ref[...] = vv / (vv * t_ref[...] + EPS)


def _final_kernel(k_ref, u_ref, v_ref, p_ref):
    p_ref[...] = (
        k_ref[...].astype(jnp.float32) * u_ref[:, 0:1] * v_ref[...]
    )


def kernel(weight):
    f32 = jnp.float32
    nb_io = N // BR_IO
    nb = N // BR

    k_mat, u, v = pl.pallas_call(
        _first_kernel,
        grid=(nb_io,),
        in_specs=[pl.BlockSpec((BR_IO, N), lambda i: (i, 0))],
        out_specs=[
            pl.BlockSpec((BR_IO, N), lambda i: (i, 0)),
            pl.BlockSpec((BR_IO, 128), lambda i: (i, 0)),
            pl.BlockSpec((1, N), lambda i: (0, 0)),
        ],
        out_shape=[
            jax.ShapeDtypeStruct((N, N), jnp.bfloat16),
            jax.ShapeDtypeStruct((N, 128), f32),
            jax.ShapeDtypeStruct((1, N), f32),
        ],
        scratch_shapes=[pltpu.VMEM((1, N), f32)],
        compiler_params=pltpu.CompilerParams(
            dimension_semantics=("arbitrary",),
            vmem_limit_bytes=VMEM_LIMIT,
        ),
        name="sinkhorn_first",
    )(weight)

    iter_call = pl.pallas_call(
        _iter_kernel,
        grid=(nb,),
        in_specs=[
            pl.BlockSpec((BR, N), lambda i: (i, 0)),
            pl.BlockSpec((BR, 128), lambda i: (i, 0)),
            pl.BlockSpec((1, N), lambda i: (0, 0)),
        ],
        out_specs=[
            pl.BlockSpec((BR, 128), lambda i: (i, 0)),
            pl.BlockSpec((1, N), lambda i: (0, 0)),
        ],
        out_shape=[
            jax.ShapeDtypeStruct((N, 128), f32),
            jax.ShapeDtypeStruct((1, N), f32),
        ],
        scratch_shapes=[pltpu.VMEM((1, N), f32)],
        compiler_params=pltpu.CompilerParams(
            dimension_semantics=("arbitrary",),
            vmem_limit_bytes=VMEM_LIMIT,
        ),
        name="sinkhorn_iter",
    )

    for _ in range(9):
        u, v = iter_call(k_mat, u, v)

    return pl.pallas_call(
        _final_kernel,
        grid=(nb_io,),
        in_specs=[
            pl.BlockSpec((BR_IO, N), lambda i: (i, 0)),
            pl.BlockSpec((BR_IO, 128), lambda i: (i, 0)),
            pl.BlockSpec((1, N), lambda i: (0, 0)),
        ],
        out_specs=pl.BlockSpec((BR_IO, N), lambda i: (i, 0)),
        out_shape=jax.ShapeDtypeStruct((N, N), f32),
        scratch_shapes=[],
        compiler_params=pltpu.CompilerParams(
            dimension_semantics=("arbitrary",),
            vmem_limit_bytes=VMEM_LIMIT,
        ),
        name="sinkhorn_final",
    )(k_mat, u, v)


# speedup vs baseline: 4.1697x; 1.0547x over previous
import jax
import jax.numpy as jnp
from jax.experimental import pallas as pl
from jax.experimental.pallas import tpu as pltpu

N = 8192
EPS = 1e-8
BR_IO = 256
BR = 512
VMEM_LIMIT = 56 * 1024 * 1024


def _first_kernel(w_ref, k_ref, un_ref, vn_ref, t_ref):
    i = pl.program_id(0)

    @pl.when(i == 0)
    def _():
        t_ref[...] = jnp.zeros_like(t_ref)

    kb = jnp.exp(w_ref[...])
    k_ref[...] = kb.astype(jnp.bfloat16)
    s = jnp.sum(kb, axis=1, keepdims=True)
    un = 1.0 / (s + EPS)
    un_ref[...] = jnp.broadcast_to(un, (BR_IO, 128))
    t_ref[...] += jnp.sum(kb * un, axis=0, keepdims=True)

    @pl.when(i == N // BR_IO - 1)
    def _():
        vn_ref[...] = 1.0 / (t_ref[...] + EPS)


NB = N // BR
NITER = 9


def _iters_kernel(k_ref, u_ref, v_ref, un_ref, vn_ref,
                  u_scr, v_scr, t_ref):
    j = pl.program_id(0)
    i = pl.program_id(1)

    @pl.when(jnp.logical_and(j == 0, i == 0))
    def _():
        v_scr[...] = v_ref[...]

    @pl.when(i == 0)
    def _():
        t_ref[...] = jnp.zeros_like(t_ref)

    kb16 = k_ref[...]
    v16 = v_scr[...].astype(jnp.bfloat16)
    r = kb16 * v16
    r = r[:, : N // 2] + r[:, N // 2 :]
    r = r[:, : N // 4] + r[:, N // 4 :]
    r = r[:, : N // 8] + r[:, N // 8 :]
    s = jnp.sum(r.astype(jnp.float32), axis=1, keepdims=True)
    rows = pl.ds(i * BR, BR)
    u = jnp.where(j == 0, u_ref[:, 0:1], u_scr[rows, 0:1])
    un = u / (u * s + EPS)
    un_b = jnp.broadcast_to(un, (BR, 128))
    u_scr[rows, :] = un_b
    x = kb16 * un.astype(jnp.bfloat16)
    x = x[: BR // 2] + x[BR // 2 :]
    x = x[: BR // 4] + x[BR // 4 :]
    x = x[: BR // 8] + x[BR // 8 :]
    t_ref[...] += jnp.sum(
        x.astype(jnp.float32), axis=0, keepdims=True
    )

    @pl.when(i == NB - 1)
    def _():
        vv = v_scr[...]
        v_scr[...] = vv / (vv * t_ref[...] + EPS)

    @pl.when(j == NITER - 1)
    def _():
        un_ref[...] = un_b

        @pl.when(i == NB - 1)
        def _():
            vn_ref[...] = v_scr[...]


def _final_kernel(k_ref, u_ref, v_ref, p_ref):
    p_ref[...] = (
        k_ref[...].astype(jnp.float32) * u_ref[:, 0:1] * v_ref[...]
    )


def kernel(weight):
    f32 = jnp.float32
    nb_io = N // BR_IO
    nb = N // BR

    k_mat, u, v = pl.pallas_call(
        _first_kernel,
        grid=(nb_io,),
        in_specs=[pl.BlockSpec((BR_IO, N), lambda i: (i, 0))],
        out_specs=[
            pl.BlockSpec((BR_IO, N), lambda i: (i, 0)),
            pl.BlockSpec((BR_IO, 128), lambda i: (i, 0)),
            pl.BlockSpec((1, N), lambda i: (0, 0)),
        ],
        out_shape=[
            jax.ShapeDtypeStruct((N, N), jnp.bfloat16),
            jax.ShapeDtypeStruct((N, 128), f32),
            jax.ShapeDtypeStruct((1, N), f32),
        ],
        scratch_shapes=[pltpu.VMEM((1, N), f32)],
        compiler_params=pltpu.CompilerParams(
            dimension_semantics=("arbitrary",),
            vmem_limit_bytes=VMEM_LIMIT,
        ),
        name="sinkhorn_first",
    )(weight)

    u, v = pl.pallas_call(
        _iters_kernel,
        grid=(NITER, nb),
        in_specs=[
            pl.BlockSpec((BR, N), lambda j, i: (i, 0)),
            pl.BlockSpec(
                (BR, 128), lambda j, i: (jnp.where(j == 0, i, 0), 0)
            ),
            pl.BlockSpec((1, N), lambda j, i: (0, 0)),
        ],
        out_specs=[
            pl.BlockSpec(
                (BR, 128),
                lambda j, i: (jnp.where(j == NITER - 1, i, 0), 0),
            ),
            pl.BlockSpec((1, N), lambda j, i: (0, 0)),
        ],
        out_shape=[
            jax.ShapeDtypeStruct((N, 128), f32),
            jax.ShapeDtypeStruct((1, N), f32),
        ],
        scratch_shapes=[
            pltpu.VMEM((N, 128), f32),
            pltpu.VMEM((1, N), f32),
            pltpu.VMEM((1, N), f32),
        ],
        compiler_params=pltpu.CompilerParams(
            dimension_semantics=("arbitrary", "arbitrary"),
            vmem_limit_bytes=VMEM_LIMIT,
        ),
        name="sinkhorn_iters",
    )(k_mat, u, v)

    return pl.pallas_call(
        _final_kernel,
        grid=(nb_io,),
        in_specs=[
            pl.BlockSpec((BR_IO, N), lambda i: (i, 0)),
            pl.BlockSpec((BR_IO, 128), lambda i: (i, 0)),
            pl.BlockSpec((1, N), lambda i: (0, 0)),
        ],
        out_specs=pl.BlockSpec((BR_IO, N), lambda i: (i, 0)),
        out_shape=jax.ShapeDtypeStruct((N, N), f32),
        compiler_params=pltpu.CompilerParams(
            dimension_semantics=("arbitrary",),
            vmem_limit_bytes=VMEM_LIMIT,
        ),
        name="sinkhorn_final",
    )(k_mat, u, v)
